# Initial kernel scaffold; baseline (speedup 1.0000x reference)
import functools

import jax
import jax.numpy as jnp
from jax import lax
from jax.experimental import pallas as pl
from jax.experimental.pallas import tpu as pltpu
from jax.experimental.pallas import tpu_sc as plsc

B, L = 4096, 200
D_ITEM, D_CAT, D_BRAND = 64, 32, 32
D_OUT = D_ITEM + D_CAT + D_BRAND
N = B * L

NC, NS = 2, 16
NW = NC * NS
TOK_PER_W = N // NW
C = 512
NCHUNK = TOK_PER_W // C

_MESH = plsc.VectorSubcoreMesh(core_axis_name="c", subcore_axis_name="s")


@functools.partial(
    pl.kernel,
    out_type=jax.ShapeDtypeStruct((N, D_OUT), jnp.float32),
    mesh=_MESH,
    scratch_types=[
        pltpu.VMEM((C,), jnp.int32),
        pltpu.VMEM((C,), jnp.int32),
        pltpu.VMEM((C,), jnp.int32),
        pltpu.VMEM((C, D_ITEM), jnp.float32),
        pltpu.VMEM((C, D_CAT), jnp.float32),
        pltpu.VMEM((C, D_BRAND), jnp.float32),
        pltpu.SemaphoreType.DMA,
        pltpu.SemaphoreType.DMA,
        pltpu.SemaphoreType.DMA,
    ],
)
def _embed_sc(item_r, cat_r, brand_r, ti_r, tc_r, tb_r, out_r,
              idx_i, idx_c, idx_b, rows_i, rows_c, rows_b,
              sem_i, sem_c, sem_b):
    wid = lax.axis_index("s") * NC + lax.axis_index("c")
    w_base = wid * TOK_PER_W

    def chunk(g, carry):
        base = w_base + g * C
        pltpu.sync_copy(item_r.at[pl.ds(base, C)], idx_i)
        pltpu.sync_copy(cat_r.at[pl.ds(base, C)], idx_c)
        pltpu.sync_copy(brand_r.at[pl.ds(base, C)], idx_b)
        ci = pltpu.async_copy(ti_r.at[idx_i], rows_i, sem_i)
        cc = pltpu.async_copy(tc_r.at[idx_c], rows_c, sem_c)
        cb = pltpu.async_copy(tb_r.at[idx_b], rows_b, sem_b)
        ci.wait()
        cc.wait()
        cb.wait()
        pltpu.sync_copy(rows_i, out_r.at[pl.ds(base, C), pl.ds(0, D_ITEM)])
        pltpu.sync_copy(rows_c, out_r.at[pl.ds(base, C), pl.ds(D_ITEM, D_CAT)])
        pltpu.sync_copy(rows_b, out_r.at[pl.ds(base, C), pl.ds(D_ITEM + D_CAT, D_BRAND)])
        return carry

    lax.fori_loop(0, NCHUNK, chunk, 0)


def kernel(item, category, brand, T_item, T_category, T_brand):
    item_f = item.reshape(N).astype(jnp.int32)
    cat_f = category.reshape(N).astype(jnp.int32)
    brand_f = brand.reshape(N).astype(jnp.int32)
    out = _embed_sc(item_f, cat_f, brand_f, T_item, T_category, T_brand)
    return out.reshape(B, L, D_OUT)


# speedup vs baseline: 10.4792x; 10.4792x over previous
import functools

import jax
import jax.numpy as jnp
from jax import lax
from jax.experimental import pallas as pl
from jax.experimental.pallas import tpu as pltpu
from jax.experimental.pallas import tpu_sc as plsc

B, L = 4096, 200
D_ITEM, D_CAT, D_BRAND = 64, 32, 32
D_OUT = D_ITEM + D_CAT + D_BRAND
N = B * L

NC, NS = 2, 16
NW = NC * NS
TOK_PER_W = N // NW
C = 512
NCHUNK = TOK_PER_W // C

_MESH = plsc.VectorSubcoreMesh(core_axis_name="c", subcore_axis_name="s")


@functools.partial(
    pl.kernel,
    out_type=jax.ShapeDtypeStruct((N, D_OUT), jnp.float32),
    mesh=_MESH,
    scratch_types=[
        pltpu.VMEM((C,), jnp.int32),
        pltpu.VMEM((C,), jnp.int32),
        pltpu.VMEM((C,), jnp.int32),
        pltpu.VMEM((C, D_OUT), jnp.float32),
        pltpu.SemaphoreType.DMA,
        pltpu.SemaphoreType.DMA,
        pltpu.SemaphoreType.DMA,
    ],
)
def _embed_sc(item_r, cat_r, brand_r, ti_r, tc_r, tb_r, out_r,
              idx_i, idx_c, idx_b, out_v,
              sem_i, sem_c, sem_b):
    wid = lax.axis_index("s") * NC + lax.axis_index("c")
    w_base = wid * TOK_PER_W

    def chunk(g, carry):
        base = w_base + g * C
        pltpu.sync_copy(item_r.at[pl.ds(base, C)], idx_i)
        pltpu.sync_copy(cat_r.at[pl.ds(base, C)], idx_c)
        pltpu.sync_copy(brand_r.at[pl.ds(base, C)], idx_b)
        pltpu.async_copy(ti_r.at[idx_i], out_v, sem_i).wait()
        cc = pltpu.async_copy(tc_r.at[idx_c], out_v, sem_c, add=True)
        cb = pltpu.async_copy(tb_r.at[idx_b], out_v, sem_b, add=True)
        cc.wait()
        cb.wait()
        pltpu.sync_copy(out_v, out_r.at[pl.ds(base, C)])
        return carry

    lax.fori_loop(0, NCHUNK, chunk, 0)


def kernel(item, category, brand, T_item, T_category, T_brand):
    item_f = item.reshape(N).astype(jnp.int32)
    cat_f = category.reshape(N).astype(jnp.int32)
    brand_f = brand.reshape(N).astype(jnp.int32)
    ti_p = jnp.pad(T_item, ((0, 0), (0, D_OUT - D_ITEM)))
    tc_p = jnp.pad(T_category, ((0, 0), (D_ITEM, D_OUT - D_ITEM - D_CAT)))
    tb_p = jnp.pad(T_brand, ((0, 0), (D_OUT - D_BRAND, 0)))
    out = _embed_sc(item_f, cat_f, brand_f, ti_p, tc_p, tb_p)
    return out.reshape(B, L, D_OUT)


# speedup vs baseline: 11.7137x; 1.1178x over previous
import functools

import jax
import jax.numpy as jnp
from jax import lax
from jax.experimental import pallas as pl
from jax.experimental.pallas import tpu as pltpu
from jax.experimental.pallas import tpu_sc as plsc

B, L = 4096, 200
D_ITEM, D_CAT, D_BRAND = 64, 32, 32
D_OUT = D_ITEM + D_CAT + D_BRAND
N = B * L

NC, NS = 2, 16
NW = NC * NS
TOK_PER_W = N // NW
C = 256
NPAIR = TOK_PER_W // (2 * C)

_MESH = plsc.VectorSubcoreMesh(core_axis_name="c", subcore_axis_name="s")


@functools.partial(
    pl.kernel,
    out_type=jax.ShapeDtypeStruct((N, D_OUT), jnp.float32),
    mesh=_MESH,
    scratch_types=[
        pltpu.VMEM((C,), jnp.int32),
        pltpu.VMEM((C,), jnp.int32),
        pltpu.VMEM((C,), jnp.int32),
        pltpu.VMEM((C,), jnp.int32),
        pltpu.VMEM((C,), jnp.int32),
        pltpu.VMEM((C,), jnp.int32),
        pltpu.VMEM((C, D_OUT), jnp.float32),
        pltpu.VMEM((C, D_OUT), jnp.float32),
        pltpu.SemaphoreType.DMA,
        pltpu.SemaphoreType.DMA,
        pltpu.SemaphoreType.DMA,
        pltpu.SemaphoreType.DMA,
        pltpu.SemaphoreType.DMA,
        pltpu.SemaphoreType.DMA,
        pltpu.SemaphoreType.DMA,
        pltpu.SemaphoreType.DMA,
    ],
)
def _embed_sc(item_r, cat_r, brand_r, ti_r, tc_r, tb_r, out_r,
              idx_ia, idx_ca, idx_ba, idx_ib, idx_cb, idx_bb,
              out_va, out_vb,
              sem_ia, sem_ib, sem_ga, sem_gb, sem_aa, sem_ab, sem_wa, sem_wb):
    wid = lax.axis_index("s") * NC + lax.axis_index("c")
    w_base = wid * TOK_PER_W

    def do_pair(base_a, wait_writes):
        base_b = base_a + C
        if wait_writes:
            pltpu.make_async_copy(out_va, out_r.at[pl.ds(base_a, C)], sem_wa).wait()
            pltpu.make_async_copy(out_vb, out_r.at[pl.ds(base_b, C)], sem_wb).wait()

        i1 = pltpu.async_copy(item_r.at[pl.ds(base_a, C)], idx_ia, sem_ia)
        i2 = pltpu.async_copy(cat_r.at[pl.ds(base_a, C)], idx_ca, sem_ia)
        i3 = pltpu.async_copy(brand_r.at[pl.ds(base_a, C)], idx_ba, sem_ia)
        i4 = pltpu.async_copy(item_r.at[pl.ds(base_b, C)], idx_ib, sem_ib)
        i5 = pltpu.async_copy(cat_r.at[pl.ds(base_b, C)], idx_cb, sem_ib)
        i6 = pltpu.async_copy(brand_r.at[pl.ds(base_b, C)], idx_bb, sem_ib)

        i1.wait()
        i2.wait()
        i3.wait()
        ga = pltpu.async_copy(ti_r.at[idx_ia], out_va, sem_ga)
        i4.wait()
        i5.wait()
        i6.wait()
        gb = pltpu.async_copy(ti_r.at[idx_ib], out_vb, sem_gb)

        ga.wait()
        aa1 = pltpu.async_copy(tc_r.at[idx_ca], out_va, sem_aa, add=True)
        aa2 = pltpu.async_copy(tb_r.at[idx_ba], out_va, sem_aa, add=True)
        gb.wait()
        ab1 = pltpu.async_copy(tc_r.at[idx_cb], out_vb, sem_ab, add=True)
        ab2 = pltpu.async_copy(tb_r.at[idx_bb], out_vb, sem_ab, add=True)

        aa1.wait()
        aa2.wait()
        pltpu.async_copy(out_va, out_r.at[pl.ds(base_a, C)], sem_wa)
        ab1.wait()
        ab2.wait()
        pltpu.async_copy(out_vb, out_r.at[pl.ds(base_b, C)], sem_wb)

    do_pair(w_base, wait_writes=False)

    def pair(g, carry):
        do_pair(w_base + (2 * g) * C, wait_writes=True)
        return carry

    lax.fori_loop(1, NPAIR, pair, 0)

    last = w_base + (2 * NPAIR - 2) * C
    pltpu.make_async_copy(out_va, out_r.at[pl.ds(last, C)], sem_wa).wait()
    pltpu.make_async_copy(out_vb, out_r.at[pl.ds(last + C, C)], sem_wb).wait()


def kernel(item, category, brand, T_item, T_category, T_brand):
    item_f = item.reshape(N).astype(jnp.int32)
    cat_f = category.reshape(N).astype(jnp.int32)
    brand_f = brand.reshape(N).astype(jnp.int32)
    ti_p = jnp.pad(T_item, ((0, 0), (0, D_OUT - D_ITEM)))
    tc_p = jnp.pad(T_category, ((0, 0), (D_ITEM, D_OUT - D_ITEM - D_CAT)))
    tb_p = jnp.pad(T_brand, ((0, 0), (D_OUT - D_BRAND, 0)))
    out = _embed_sc(item_f, cat_f, brand_f, ti_p, tc_p, tb_p)
    return out.reshape(B, L, D_OUT)
